# Initial kernel scaffold; baseline (speedup 1.0000x reference)
import jax
import jax.numpy as jnp
from jax.experimental import pallas as pl


def kernel(node_features, edge_indices, edge_features, layers, heads):
    raise NotImplementedError("write your pallas kernel here")


# speedup vs baseline: 2.0276x; 2.0276x over previous
import functools

import jax
import jax.numpy as jnp
from jax import lax
from jax.experimental import pallas as pl
from jax.experimental.pallas import tpu as pltpu

N = 256
E = 16384
H = 128
L = 4
C = 2048
NB = E // C
F32 = jnp.float32
_xdot = functools.partial(jnp.dot, preferred_element_type=jnp.float32,
                          precision=jax.lax.Precision.HIGHEST)


def _bdot(a, b):
    return jnp.dot(a.astype(jnp.bfloat16), b.astype(jnp.bfloat16),
                   preferred_element_type=jnp.float32)


def _prep_body(ef_ref, dstT_ref, x0_ref, wet_ref, wm1b_ref, bet_ref, bm1_ref,
               wnt0_ref, bnt0_ref, wm1a0_ref,
               q0_ref, q1_ref, q2_ref, q3_ref, cnt_ref, p0_ref):
    i = pl.program_id(0)
    ef = ef_ref[0]
    for l, qref in enumerate((q0_ref, q1_ref, q2_ref, q3_ref)):
        ee = _bdot(ef, wet_ref[l]) + bet_ref[l]
        qref[0] = _bdot(ee, wm1b_ref[l]) + bm1_ref[l]

    oh_dst_t = (dstT_ref[0] == lax.broadcasted_iota(jnp.int32, (N, C), 0)).astype(F32)
    chunk_cnt = jnp.sum(oh_dst_t, axis=1, keepdims=True)

    @pl.when(i == 0)
    def _():
        cnt_ref[...] = chunk_cnt
        ne = _bdot(x0_ref[...], wnt0_ref[...]) + bnt0_ref[...]
        p0_ref[...] = _bdot(ne, wm1a0_ref[...])

    @pl.when(i > 0)
    def _():
        cnt_ref[...] += chunk_cnt


def _prep(ef3, dstT3, x0, wet_s, wm1b_s, bet_s, bm1_s, wnt0, bnt0, wm1a0):
    full = lambda shp: pl.BlockSpec(shp, lambda i: (0,) * len(shp))
    return pl.pallas_call(
        _prep_body,
        grid=(NB,),
        in_specs=[
            pl.BlockSpec((1, C, 4), lambda i: (i, 0, 0)),
            pl.BlockSpec((1, 1, C), lambda i: (i, 0, 0)),
            full((N, 14)), full((L, 4, H)), full((L, H, H)), full((L, 1, H)),
            full((L, 1, H)), full((14, H)), full((1, H)), full((H, H)),
        ],
        out_specs=[
            pl.BlockSpec((1, C, H), lambda i: (i, 0, 0)),
            pl.BlockSpec((1, C, H), lambda i: (i, 0, 0)),
            pl.BlockSpec((1, C, H), lambda i: (i, 0, 0)),
            pl.BlockSpec((1, C, H), lambda i: (i, 0, 0)),
            full((N, 1)), full((N, H)),
        ],
        out_shape=[
            jax.ShapeDtypeStruct((NB, C, H), F32),
            jax.ShapeDtypeStruct((NB, C, H), F32),
            jax.ShapeDtypeStruct((NB, C, H), F32),
            jax.ShapeDtypeStruct((NB, C, H), F32),
            jax.ShapeDtypeStruct((N, 1), F32),
            jax.ShapeDtypeStruct((N, H), F32),
        ],
    )(ef3, dstT3, x0, wet_s, wm1b_s, bet_s, bm1_s, wnt0, bnt0, wm1a0)


def _seg_body(p_ref, q_ref, src_ref, dstT_ref, wm2_ref, hi_ref, lo_ref):
    i = pl.program_id(0)
    oh_src = (src_ref[0] == lax.broadcasted_iota(jnp.int32, (C, N), 1)).astype(F32)
    g = _xdot(oh_src, p_ref[...])
    r = jnp.maximum(g + q_ref[0], 0.0)
    m2 = _bdot(r, wm2_ref[...])
    mhi = (m2 * 262144.0).astype(jnp.int32).astype(F32) * 3.814697265625e-06
    mlo = m2 - mhi
    oh_dst_t = (dstT_ref[0] == lax.broadcasted_iota(jnp.int32, (N, C), 0)).astype(F32)
    hi_upd = _xdot(oh_dst_t, mhi)
    lo_upd = _xdot(oh_dst_t, mlo)

    @pl.when(i == 0)
    def _():
        hi_ref[...] = hi_upd
        lo_ref[...] = lo_upd

    @pl.when(i > 0)
    def _():
        hi_ref[...] += hi_upd
        lo_ref[...] += lo_upd


def _segment_sum(p, q3, src3, dstT3, wm2):
    hi, lo = pl.pallas_call(
        _seg_body,
        grid=(NB,),
        in_specs=[
            pl.BlockSpec((N, H), lambda i: (0, 0)),
            pl.BlockSpec((1, C, H), lambda i: (i, 0, 0)),
            pl.BlockSpec((1, C, 1), lambda i: (i, 0, 0)),
            pl.BlockSpec((1, 1, C), lambda i: (i, 0, 0)),
            pl.BlockSpec((H, H), lambda i: (0, 0)),
        ],
        out_specs=[pl.BlockSpec((N, H), lambda i: (0, 0)),
                   pl.BlockSpec((N, H), lambda i: (0, 0))],
        out_shape=[jax.ShapeDtypeStruct((N, H), F32),
                   jax.ShapeDtypeStruct((N, H), F32)],
    )(p, q3, src3, dstT3, wm2)
    return jnp.stack([hi, lo])


def _row_sum_xla(x):
    acc = x[:, 0:8]
    for j in range(1, 16):
        acc = acc + x[:, 8 * j:8 * j + 8]
    a = acc[:, 0:4] + acc[:, 4:8]
    b = a[:, 0:2] + a[:, 2:4]
    return b[:, 0:1] + b[:, 1:2]


def _update_body(has_next, residual, nparts, refs):
    (x_ref, parts_ref, cnt_ref, bm2_ref, wu1a_ref, wu1b_ref, bu1_ref,
     wu2_ref, bu2_ref, lng_ref, lnb_ref) = refs[:11]
    k = 11
    if has_next:
        wntn_ref, bntn_ref, wm1an_ref = refs[k:k + 3]
        k += 3
    xn_ref = refs[k]
    if has_next:
        pn_ref = refs[k + 1]

    agg = parts_ref[0]
    for j in range(1, nparts):
        agg = agg + parts_ref[j]
    aggf = agg + cnt_ref[...] * bm2_ref[...]
    u = jnp.maximum(
        _bdot(x_ref[...], wu1a_ref[...])
        + _bdot(aggf, wu1b_ref[...]) + bu1_ref[...], 0.0)
    out = _bdot(u, wu2_ref[...]) + bu2_ref[...]
    mu = _row_sum_xla(out) * (1.0 / H)
    d = out - mu
    var = _row_sum_xla(d * d) * (1.0 / H)
    outn = d / jnp.sqrt(var + 1e-5) * lng_ref[...] + lnb_ref[...]
    if residual:
        outn = outn + x_ref[...]
    xn = jnp.maximum(outn, 0.0)
    xn_ref[...] = xn
    if has_next:
        ne = _bdot(xn, wntn_ref[...]) + bntn_ref[...]
        pn_ref[...] = _bdot(ne, wm1an_ref[...])


def _update(x, parts, cnt, bm2, wu1a, wu1b, bu1, wu2, bu2, lng, lnb,
            nxt=None):
    D = x.shape[1]
    has_next = nxt is not None
    nparts = parts.shape[0]
    ins = [x, parts, cnt, bm2, wu1a, wu1b, bu1, wu2, bu2, lng, lnb]
    if has_next:
        ins += list(nxt)
    out_shape = [jax.ShapeDtypeStruct((N, H), F32)]
    if has_next:
        out_shape.append(jax.ShapeDtypeStruct((N, H), F32))
    body = lambda *refs: _update_body(has_next, D == H, nparts, refs)
    res = pl.pallas_call(body, out_shape=out_shape)(*ins)
    return (res[0], res[1]) if has_next else (res[0], None)


BI = 16


def _head_body(x_ref, xb_ref, mw1_ref, mb1_ref, mw2_ref, mb2_ref, mw3_ref, mb3_ref,
               sw1_ref, sb1_ref, sw2_ref, sb2_ref, sw3t_ref, sb3_ref,
               tw1a_ref, tw1b_ref, tb1_ref, tw2_ref, tb2_ref, tw3t_ref, tb3_ref,
               mvl_ref, mvp_ref, sh_ref, tst_ref, bscr):
    i = pl.program_id(0)

    @pl.when(i == 0)
    def _():
        x = x_ref[...]
        bscr[...] = _bdot(x, tw1b_ref[...])
        h1 = jnp.maximum(_bdot(x, mw1_ref[...]) + mb1_ref[...], 0.0)
        h2 = jnp.maximum(_bdot(h1, mw2_ref[...]) + mb2_ref[...], 0.0)
        ml = _bdot(h2, mw3_ref[...]) + mb3_ref[...]
        mvl_ref[...] = ml
        ex = jnp.exp(ml - jnp.max(ml, axis=-1, keepdims=True))
        p = ex / jnp.sum(ex, axis=-1, keepdims=True)
        mvp_ref[...] = jnp.clip(p, 0.001, 0.999)
        s1 = jnp.maximum(_bdot(x, sw1_ref[...]) + sb1_ref[...], 0.0)
        s2 = jnp.maximum(_bdot(s1, sw2_ref[...]) + sb2_ref[...], 0.0)
        sv = _bdot(s2, sw3t_ref[...]) + sb3_ref[...]
        sh_ref[...] = jnp.clip(sv, -10.0, 10.0)

    a_blk = _bdot(xb_ref[...], tw1a_ref[...]) + tb1_ref[...]
    b_all = bscr[...]
    cols = []
    for rr in range(BI):
        h = jnp.maximum(a_blk[rr:rr + 1, :] + b_all, 0.0)
        t = jnp.maximum(_bdot(h, tw2_ref[...]) + tb2_ref[...], 0.0)
        s = _bdot(t, tw3t_ref[...]) + tb3_ref[...]
        cols.append(s)
    tst_ref[0] = jnp.clip(jnp.concatenate(cols, axis=1), -10.0, 10.0)


def _head(x, hd):
    full = lambda shp: pl.BlockSpec(shp, lambda i: (0,) * len(shp))
    r1 = lambda a: a.reshape(1, -1)
    ins = [
        x, x,
        hd['mv_W1'], r1(hd['mv_b1']), hd['mv_W2'], r1(hd['mv_b2']),
        hd['mv_W3'], r1(hd['mv_b3']),
        hd['sh_W1'], r1(hd['sh_b1']), hd['sh_W2'], r1(hd['sh_b2']),
        hd['sh_W3'], r1(hd['sh_b3']),
        hd['tg_W1'][:H], hd['tg_W1'][H:], r1(hd['tg_b1']),
        hd['tg_W2'], r1(hd['tg_b2']), hd['tg_W3'], r1(hd['tg_b3']),
    ]
    in_specs = [full((N, H)), pl.BlockSpec((BI, H), lambda i: (i, 0))]
    for a in ins[2:]:
        in_specs.append(full(a.shape))
    mvl, mvp, sh, tst = pl.pallas_call(
        _head_body,
        grid=(N // BI,),
        in_specs=in_specs,
        out_specs=[full((N, 4)), full((N, 4)), full((N, 1)),
                   pl.BlockSpec((1, N, BI), lambda i: (i, 0, 0))],
        out_shape=[
            jax.ShapeDtypeStruct((N, 4), F32),
            jax.ShapeDtypeStruct((N, 4), F32),
            jax.ShapeDtypeStruct((N, 1), F32),
            jax.ShapeDtypeStruct((N // BI, N, BI), F32),
        ],
        scratch_shapes=[pltpu.VMEM((N, H), F32)],
    )(*ins)
    return mvl, mvp, sh, tst


def kernel(node_features, edge_indices, edge_features, layers, heads):
    src3 = edge_indices[0].reshape(NB, C, 1)
    dstT3 = edge_indices[1].reshape(NB, 1, C)
    ef3 = edge_features.reshape(NB, C, 4)
    r1 = lambda a: a.reshape(1, -1)

    wet_s = jnp.stack([p['W_et'] for p in layers])
    wm1b_s = jnp.stack([p['W_m1'][H:] for p in layers])
    bet_s = jnp.stack([r1(p['b_et']) for p in layers])
    bm1_s = jnp.stack([r1(p['b_m1']) for p in layers])

    q0, q1, q2, q3, cnt, p_cur = _prep(
        ef3, dstT3, node_features, wet_s, wm1b_s, bet_s, bm1_s,
        layers[0]['W_nt'], r1(layers[0]['b_nt']), layers[0]['W_m1'][:H])
    qs = (q0, q1, q2, q3)

    x = node_features
    for l, p in enumerate(layers):
        D = x.shape[1]
        parts = _segment_sum(p_cur, qs[l], src3, dstT3, p['W_m2'])
        nxt = None
        if l + 1 < L:
            pn = layers[l + 1]
            nxt = (pn['W_nt'], r1(pn['b_nt']), pn['W_m1'][:H])
        x, p_cur = _update(
            x, parts, cnt, r1(p['b_m2']),
            p['W_u1'][:D], p['W_u1'][D:], r1(p['b_u1']),
            p['W_u2'], r1(p['b_u2']), r1(p['ln_g']), r1(p['ln_b']), nxt=nxt)

    mvl, mvp, sh, tst = _head(x, heads)
    ts = tst.transpose(0, 2, 1).reshape(N, N)
    target_scores = ts.reshape(-1)[1:].reshape(N - 1, N + 1)[:, :-1].reshape(-1)
    return (mvl, mvp, sh.reshape(N), target_scores, x)
